# Initial kernel scaffold; baseline (speedup 1.0000x reference)
import jax
import jax.numpy as jnp
from jax.experimental import pallas as pl


def kernel(pos):
    raise NotImplementedError("write your pallas kernel here")


# speedup vs baseline: 3.2658x; 3.2658x over previous
import jax
import jax.numpy as jnp
from jax.experimental import pallas as pl

_N = 8192
_K = 32
_QT = 128
_CUTOFF = 6.0


def _knn_kernel(kpos_ref, qposT_ref, qpos_ref, dist_ref, src_ref, dst_ref):
    q = pl.program_id(0)
    kx = kpos_ref[:, 0:1]
    ky = kpos_ref[:, 1:2]
    kz = kpos_ref[:, 2:3]
    qx = qposT_ref[0:1, :]
    qy = qposT_ref[1:2, :]
    qz = qposT_ref[2:3, :]
    ksq = kx * kx + ky * ky + kz * kz
    qsq = qx * qx + qy * qy + qz * qz
    dots = jax.lax.dot_general(
        kpos_ref[:, :],
        qpos_ref[:, :],
        dimension_numbers=(((1,), (1,)), ((), ())),
        preferred_element_type=jnp.float32,
    )
    d2 = (ksq + qsq) - 2.0 * dots
    d2 = jnp.maximum(d2, 0.0)
    kiota = jax.lax.broadcasted_iota(jnp.int32, (_N, _QT), 0)
    lane = jax.lax.broadcasted_iota(jnp.int32, (1, _QT), 1)
    qid = q * _QT + lane
    d2 = jnp.where(kiota == qid, 1e20, d2)
    D = jnp.sqrt(d2 + 1e-12)
    for t in range(_K):
        m = jnp.min(D, axis=0, keepdims=True)
        am = jnp.min(jnp.where(D == m, kiota, _N), axis=0, keepdims=True)
        valid = m <= _CUTOFF
        dist_ref[t : t + 1, :] = jnp.where(valid, m, 0.0)
        src_ref[t : t + 1, :] = jnp.where(valid, am, -1)
        dst_ref[t : t + 1, :] = jnp.where(valid, qid, -1)
        if t < _K - 1:
            D = jnp.where(kiota == am, jnp.float32(3e20), D)


def kernel(pos):
    posT = pos.T
    grid = _N // _QT
    dist_o, src_o, dst_o = pl.pallas_call(
        _knn_kernel,
        grid=(grid,),
        in_specs=[
            pl.BlockSpec((_N, 3), lambda i: (0, 0)),
            pl.BlockSpec((3, _QT), lambda i: (0, i)),
            pl.BlockSpec((_QT, 3), lambda i: (i, 0)),
        ],
        out_specs=[
            pl.BlockSpec((_K, _QT), lambda i: (0, i)),
            pl.BlockSpec((_K, _QT), lambda i: (0, i)),
            pl.BlockSpec((_K, _QT), lambda i: (0, i)),
        ],
        out_shape=[
            jax.ShapeDtypeStruct((_K, _N), jnp.float32),
            jax.ShapeDtypeStruct((_K, _N), jnp.int32),
            jax.ShapeDtypeStruct((_K, _N), jnp.int32),
        ],
    )(pos, posT, pos)
    nbr_dist = dist_o.T
    edge_index = jnp.stack([src_o.T.reshape(-1), dst_o.T.reshape(-1)], axis=0)
    return edge_index, nbr_dist


# speedup vs baseline: 4.7040x; 1.4404x over previous
import jax
import jax.numpy as jnp
from jax.experimental import pallas as pl

_N = 8192
_K = 32
_QT = 128
_CUTOFF = 6.0


def _knn_kernel(kpos_ref, qposT_ref, qpos_ref, dist_ref, src_ref, dst_ref):
    q = pl.program_id(0)
    kx = kpos_ref[:, 0:1]
    ky = kpos_ref[:, 1:2]
    kz = kpos_ref[:, 2:3]
    qx = qposT_ref[0:1, :]
    qy = qposT_ref[1:2, :]
    qz = qposT_ref[2:3, :]
    ksq = kx * kx + ky * ky + kz * kz
    qsq = qx * qx + qy * qy + qz * qz
    dots = jax.lax.dot_general(
        kpos_ref[:, :],
        qpos_ref[:, :],
        dimension_numbers=(((1,), (1,)), ((), ())),
        preferred_element_type=jnp.float32,
    )
    d2 = (ksq + qsq) - 2.0 * dots
    d2 = jnp.maximum(d2, 0.0)
    kiota = jax.lax.broadcasted_iota(jnp.int32, (_N, _QT), 0)
    lane = jax.lax.broadcasted_iota(jnp.int32, (1, _QT), 1)
    qid = q * _QT + lane
    d2 = jnp.where(kiota == qid, 1e20, d2)
    D = jnp.sqrt(d2 + 1e-12)

    G = 8
    S = _N // G
    Qs = [D[j * S : (j + 1) * S, :] for j in range(G)]
    siota = jax.lax.broadcasted_iota(jnp.int32, (S, _QT), 0)
    Is = [siota + j * S for j in range(G)]
    pairs_even = [(0, 1), (2, 3), (4, 5), (6, 7)]
    pairs_odd = [(1, 2), (3, 4), (5, 6)]
    for r in range(G):
        for (x, y) in pairs_even if r % 2 == 0 else pairs_odd:
            swap = Qs[x] > Qs[y]
            lo = jnp.minimum(Qs[x], Qs[y])
            hi = jnp.maximum(Qs[x], Qs[y])
            ilo = jnp.where(swap, Is[y], Is[x])
            ihi = jnp.where(swap, Is[x], Is[y])
            Qs[x], Qs[y] = lo, hi
            Is[x], Is[y] = ilo, ihi

    BIGI = jnp.int32(_N)
    for t in range(_K):
        m = jnp.min(Qs[0], axis=0, keepdims=True)
        e = jnp.min(jnp.where(Qs[0] == m, Is[0], BIGI), axis=0, keepdims=True)
        valid = m <= _CUTOFF
        dist_ref[t : t + 1, :] = jnp.where(valid, m, 0.0)
        src_ref[t : t + 1, :] = jnp.where(valid, e, -1)
        dst_ref[t : t + 1, :] = jnp.where(valid, qid, -1)
        if t < _K - 1:
            cond = Is[0] == e
            for j in range(G - 1):
                Qs[j] = jnp.where(cond, Qs[j + 1], Qs[j])
                Is[j] = jnp.where(cond, Is[j + 1], Is[j])
            Qs[G - 1] = jnp.where(cond, jnp.float32(3e20), Qs[G - 1])


def kernel(pos):
    posT = pos.T
    grid = _N // _QT
    dist_o, src_o, dst_o = pl.pallas_call(
        _knn_kernel,
        grid=(grid,),
        in_specs=[
            pl.BlockSpec((_N, 3), lambda i: (0, 0)),
            pl.BlockSpec((3, _QT), lambda i: (0, i)),
            pl.BlockSpec((_QT, 3), lambda i: (i, 0)),
        ],
        out_specs=[
            pl.BlockSpec((_K, _QT), lambda i: (0, i)),
            pl.BlockSpec((_K, _QT), lambda i: (0, i)),
            pl.BlockSpec((_K, _QT), lambda i: (0, i)),
        ],
        out_shape=[
            jax.ShapeDtypeStruct((_K, _N), jnp.float32),
            jax.ShapeDtypeStruct((_K, _N), jnp.int32),
            jax.ShapeDtypeStruct((_K, _N), jnp.int32),
        ],
    )(pos, posT, pos)
    nbr_dist = dist_o.T
    edge_index = jnp.stack([src_o.T.reshape(-1), dst_o.T.reshape(-1)], axis=0)
    return edge_index, nbr_dist


# speedup vs baseline: 16.9753x; 3.6087x over previous
import jax
import jax.numpy as jnp
from jax.experimental import pallas as pl

_N = 8192
_K = 32
_QT = 128
_CUTOFF = 6.0


def _knn_kernel(kpos_ref, qposT_ref, qpos_ref, dist_ref, src_ref, dst_ref):
    q = pl.program_id(0)
    kx = kpos_ref[:, 0:1]
    ky = kpos_ref[:, 1:2]
    kz = kpos_ref[:, 2:3]
    qx = qposT_ref[0:1, :]
    qy = qposT_ref[1:2, :]
    qz = qposT_ref[2:3, :]
    ksq = kx * kx + ky * ky + kz * kz
    qsq = qx * qx + qy * qy + qz * qz
    dots = jax.lax.dot_general(
        kpos_ref[:, :],
        qpos_ref[:, :],
        dimension_numbers=(((1,), (1,)), ((), ())),
        preferred_element_type=jnp.float32,
    )
    d2 = (ksq + qsq) - 2.0 * dots
    d2 = jnp.maximum(d2, 0.0)
    kiota = jax.lax.broadcasted_iota(jnp.int32, (_N, _QT), 0)
    lane = jax.lax.broadcasted_iota(jnp.int32, (1, _QT), 1)
    qid = q * _QT + lane
    d2 = jnp.where(kiota == qid, 1e20, d2)
    D = jnp.sqrt(d2 + 1e-12)

    NS = _K
    SR = _N // NS
    Q = [D[j * SR : (j + 1) * SR, :] for j in range(NS)]
    siota = jax.lax.broadcasted_iota(jnp.int32, (SR, _QT), 0)
    I = [siota + j * SR for j in range(NS)]

    def cx(i, j):
        swap = Q[j] < Q[i]
        lo = jnp.minimum(Q[i], Q[j])
        hi = jnp.maximum(Q[i], Q[j])
        I[i], I[j] = jnp.where(swap, I[j], I[i]), jnp.where(swap, I[i], I[j])
        Q[i], Q[j] = lo, hi

    k = 2
    while k <= NS:
        s = k // 2
        while s >= 1:
            for i in range(NS):
                p = i ^ s
                if p > i:
                    if (i & k) == 0:
                        cx(i, p)
                    else:
                        cx(p, i)
            s //= 2
        k *= 2

    rows = SR
    while rows > 1:
        half = rows // 2
        A = [qv[:half] for qv in Q]
        B = [qv[half:] for qv in Q]
        IA = [iv[:half] for iv in I]
        IB = [iv[half:] for iv in I]
        Qm, Im = [], []
        for j in range(NS):
            b = NS - 1 - j
            swap = B[b] < A[j]
            Qm.append(jnp.minimum(A[j], B[b]))
            Im.append(jnp.where(swap, IB[b], IA[j]))
        Q, I = Qm, Im
        for s in (16, 8, 4, 2, 1):
            for i in range(NS):
                p = i ^ s
                if p > i:
                    cx(i, p)
        rows = half

    for t in range(NS):
        m = Q[t]
        valid = m <= _CUTOFF
        dist_ref[t : t + 1, :] = jnp.where(valid, m, 0.0)
        src_ref[t : t + 1, :] = jnp.where(valid, I[t], -1)
        dst_ref[t : t + 1, :] = jnp.where(valid, qid, -1)


def kernel(pos):
    posT = pos.T
    grid = _N // _QT
    dist_o, src_o, dst_o = pl.pallas_call(
        _knn_kernel,
        grid=(grid,),
        in_specs=[
            pl.BlockSpec((_N, 3), lambda i: (0, 0)),
            pl.BlockSpec((3, _QT), lambda i: (0, i)),
            pl.BlockSpec((_QT, 3), lambda i: (i, 0)),
        ],
        out_specs=[
            pl.BlockSpec((_K, _QT), lambda i: (0, i)),
            pl.BlockSpec((_K, _QT), lambda i: (0, i)),
            pl.BlockSpec((_K, _QT), lambda i: (0, i)),
        ],
        out_shape=[
            jax.ShapeDtypeStruct((_K, _N), jnp.float32),
            jax.ShapeDtypeStruct((_K, _N), jnp.int32),
            jax.ShapeDtypeStruct((_K, _N), jnp.int32),
        ],
    )(pos, posT, pos)
    nbr_dist = dist_o.T
    edge_index = jnp.stack([src_o.T.reshape(-1), dst_o.T.reshape(-1)], axis=0)
    return edge_index, nbr_dist
